# baseline (speedup 1.0000x reference)
import functools

import jax
import jax.numpy as jnp
from jax import lax
from jax.experimental import pallas as pl
from jax.experimental.pallas import tpu as pltpu
from jax.experimental.pallas import tpu_sc as plsc

NUM_EMB = 1000000
DIM = 32
BATCH = 16384
HIST = 200
NW = 32
HT = HIST // 8
BT = BATCH // 128
UNITS = HT * BT * 2
U_PER_W = UNITS // NW
ROWS = 512


def _emb_body(idx5_hbm, table_hbm, out5_hbm, idx_v, rows_v0, rows_v1,
              tile_v0, tile_v1, sem_i, sem_g, sem_o):
    rows_ring = (rows_v0, rows_v1)
    tile_ring = (tile_v0, tile_v1)
    c = lax.axis_index("c")
    s = lax.axis_index("s")
    wid = s * 2 + c
    u0 = wid * U_PER_W

    def decode(u):
        return u // 256, (u // 2) % 128, u % 2

    def idx_copy(u, ib):
        ht, bt, half = decode(u)
        return pltpu.make_async_copy(
            idx5_hbm.at[ht, bt, pl.ds(half * ROWS, ROWS)],
            idx_v.at[ib], sem_i.at[ib])

    def gather(ib, rb):
        return pltpu.make_async_copy(
            table_hbm.at[idx_v.at[ib]], rows_ring[rb], sem_g.at[rb])

    def writeback(u, rb):
        ht, bt, half = decode(u)
        return pltpu.make_async_copy(
            tile_ring[rb].at[:, :, :, pl.ds(0, 128)],
            out5_hbm.at[pl.ds(ht * 8 + half * 4, 4), :, bt],
            sem_o.at[rb])

    iota16 = lax.iota(jnp.int32, 16)
    zero16 = jnp.zeros((16,), jnp.int32)
    dtv = [(iota16 + q * 16) // 8 for q in range(2)]
    ddv = [(iota16 + q * 16) % 8 for q in range(2)]

    def transpose(rb):
        @plsc.parallel_loop(0, 128, unroll=4)
        def _(bb):
            bbv = zero16 + bb
            for hh in range(4):
                for q in range(2):
                    v = rows_ring[rb][hh * 128 + bb, pl.ds(q * 16, 16)]
                    plsc.store_scatter(tile_ring[rb].at[hh],
                                       [dtv[q], ddv[q], bbv], v)

    idx_copy(u0, 0).start()
    idx_copy(u0 + 1, 1).start()
    idx_copy(u0, 0).wait()
    gather(0, 0).start()

    def body(p, carry):
        for b in (0, 1):
            i = p * 2 + b
            u = u0 + i
            nb = 1 - b

            @pl.when(i + 1 < U_PER_W)
            def _():
                idx_copy(u + 1, lax.rem(i + 1, 4)).wait()
                gather(lax.rem(i + 1, 4), nb).start()

            @pl.when(i + 2 < U_PER_W)
            def _():
                idx_copy(u + 2, lax.rem(i + 2, 4)).start()

            gather(lax.rem(i, 4), b).wait()

            @pl.when(i >= 2)
            def _():
                writeback(u - 2, b).wait()

            transpose(b)
            writeback(u, b).start()
        return carry

    lax.fori_loop(0, U_PER_W // 2, body, 0)
    for i in (U_PER_W - 2, U_PER_W - 1):
        writeback(u0 + i, i % 2).wait()


TK = 800
TBLK = NUM_EMB // TK
TK_G = TK // 16


def _tr_body(wt_hbm, out_hbm, w0, w1, t0, t1, sem_i, sem_o):
    c = lax.axis_index("c")
    s = lax.axis_index("s")
    wid = s * 2 + c
    wring = (w0, w1)
    tring = (t0, t1)

    def din(k, b):
        return pltpu.make_async_copy(
            wt_hbm.at[:, pl.ds((k * NW + wid) * TK, TK)], wring[b],
            sem_i.at[b])

    def dout(k, b):
        return pltpu.make_async_copy(
            tring[b].at[:, pl.ds(0, DIM)],
            out_hbm.at[pl.ds((k * NW + wid) * TK, TK), :], sem_o.at[b])

    iota16 = lax.iota(jnp.int32, 16)
    dsplat = [jnp.full((16,), d, jnp.int32) for d in range(DIM)]

    def transpose(rb):
        @plsc.parallel_loop(0, TK_G, unroll=2)
        def _(g):
            rvec = iota16 + g * 16
            for d in range(DIM):
                v = wring[rb][d, pl.ds(g * 16, 16)]
                plsc.store_scatter(tring[rb], [rvec, dsplat[d]], v)

    n_full = TBLK // NW
    has_tail = 39 * NW + wid < TBLK

    din(0, 0).start()
    din(1, 1).start()

    def body(p, carry):
        for b in (0, 1):
            k = p * 2 + b
            din(k, b).wait()

            @pl.when(p >= 1)
            def _():
                dout(k - 2, b).wait()

            transpose(b)
            dout(k, b).start()

            nk = k + 2
            @pl.when((nk < n_full) | ((nk == n_full) & has_tail))
            def _():
                din(nk, b).start()
        return carry

    lax.fori_loop(0, 19, body, 0)

    k = n_full - 1
    din(k, 0).wait()
    dout(k - 2, 0).wait()
    transpose(0)
    dout(k, 0).start()

    @pl.when(has_tail)
    def _():
        din(39, 1).wait()
        dout(37, 1).wait()
        transpose(1)
        dout(39, 1).start()

    @pl.when(jnp.logical_not(has_tail))
    def _():
        dout(37, 1).wait()

    dout(38, 0).wait()

    @pl.when(has_tail)
    def _():
        dout(39, 1).wait()


@jax.jit
def _table_rows_sc(wt):
    mesh = plsc.VectorSubcoreMesh(core_axis_name="c", subcore_axis_name="s")
    f = pl.kernel(
        _tr_body,
        mesh=mesh,
        out_type=jax.ShapeDtypeStruct((NUM_EMB, DIM), jnp.float32),
        scratch_types=[
            pltpu.VMEM((DIM, TK), jnp.float32),
            pltpu.VMEM((DIM, TK), jnp.float32),
            pltpu.VMEM((TK, DIM + 1), jnp.float32),
            pltpu.VMEM((TK, DIM + 1), jnp.float32),
            pltpu.SemaphoreType.DMA((2,)),
            pltpu.SemaphoreType.DMA((2,)),
        ],
        compiler_params=pltpu.CompilerParams(use_tc_tiling_on_sc=False,
                                             needs_layout_passes=False),
    )
    return f(wt)


@jax.jit
def _embedding_sc(idx5, weight_rows):
    mesh = plsc.VectorSubcoreMesh(core_axis_name="c", subcore_axis_name="s")
    f = pl.kernel(
        _emb_body,
        mesh=mesh,
        out_type=jax.ShapeDtypeStruct((HIST, 4, BT, 8, 128), jnp.float32),
        scratch_types=[
            pltpu.VMEM((4, ROWS), jnp.int32),
            pltpu.VMEM((ROWS, DIM), jnp.float32),
            pltpu.VMEM((ROWS, DIM), jnp.float32),
            pltpu.VMEM((4, 4, 8, 129), jnp.float32),
            pltpu.VMEM((4, 4, 8, 129), jnp.float32),
            pltpu.SemaphoreType.DMA((4,)),
            pltpu.SemaphoreType.DMA((2,)),
            pltpu.SemaphoreType.DMA((2,)),
        ],
        compiler_params=pltpu.CompilerParams(use_tc_tiling_on_sc=False,
                                             needs_layout_passes=False),
    )
    return f(idx5, weight_rows)


def kernel(input_, weight):
    idx5 = (input_.astype(jnp.int32)
            .reshape(BT, 128, HT, 8)
            .transpose(2, 0, 3, 1)
            .reshape(HT, BT, 1024))
    w_rows = _table_rows_sc(weight.T)
    out5 = _embedding_sc(idx5, w_rows)
    return out5.transpose(2, 4, 0, 1, 3).reshape(BATCH, HIST, DIM)


# speedup vs baseline: 3.7773x; 3.7773x over previous
import functools

import jax
import jax.numpy as jnp
from jax import lax
from jax.experimental import pallas as pl
from jax.experimental.pallas import tpu as pltpu
from jax.experimental.pallas import tpu_sc as plsc

NUM_EMB = 1000000
DIM = 32
BATCH = 16384
HIST = 200
NW = 32
HT = HIST // 8
BT = BATCH // 128
UNITS = HT * BT * 2
U_PER_W = UNITS // NW
ROWS = 512


def _emb_body(idx5_hbm, table_hbm, out5_hbm, idx_v, rows_v0, rows_v1,
              tile_v0, tile_v1, sem_i, sem_g, sem_o):
    rows_ring = (rows_v0, rows_v1)
    tile_ring = (tile_v0, tile_v1)
    c = lax.axis_index("c")
    s = lax.axis_index("s")
    wid = s * 2 + c
    u0 = wid * U_PER_W

    def decode(u):
        return u // 256, (u // 2) % 128, u % 2

    def idx_copy(u, ib):
        ht, bt, half = decode(u)
        return pltpu.make_async_copy(
            idx5_hbm.at[ht, bt, pl.ds(half * ROWS, ROWS)],
            idx_v.at[ib], sem_i.at[ib])

    def gather(ib, rb):
        return pltpu.make_async_copy(
            table_hbm.at[idx_v.at[ib]], rows_ring[rb], sem_g.at[rb])

    def writeback(u, rb):
        ht, bt, half = decode(u)
        return pltpu.make_async_copy(
            tile_ring[rb].at[:, :, :, pl.ds(0, 128)],
            out5_hbm.at[pl.ds(ht * 8 + half * 4, 4), :, bt],
            sem_o.at[rb])

    iota16 = lax.iota(jnp.int32, 16)
    zero16 = jnp.zeros((16,), jnp.int32)
    dtv = [(iota16 + q * 16) // 8 for q in range(2)]
    ddv = [(iota16 + q * 16) % 8 for q in range(2)]

    def transpose(rb):
        @plsc.parallel_loop(0, 128, unroll=8)
        def _(bb):
            bbv = zero16 + bb
            for hh in range(4):
                for q in range(2):
                    v = rows_ring[rb][hh * 128 + bb, pl.ds(q * 16, 16)]
                    plsc.store_scatter(tile_ring[rb].at[hh],
                                       [dtv[q], ddv[q], bbv], v)

    idx_copy(u0, 0).start()
    idx_copy(u0 + 1, 1).start()
    idx_copy(u0, 0).wait()
    gather(0, 0).start()

    def body(p, carry):
        for b in (0, 1):
            i = p * 2 + b
            u = u0 + i
            nb = 1 - b

            @pl.when(i + 1 < U_PER_W)
            def _():
                idx_copy(u + 1, lax.rem(i + 1, 4)).wait()
                gather(lax.rem(i + 1, 4), nb).start()

            @pl.when(i + 2 < U_PER_W)
            def _():
                idx_copy(u + 2, lax.rem(i + 2, 4)).start()

            gather(lax.rem(i, 4), b).wait()

            @pl.when(i >= 2)
            def _():
                writeback(u - 2, b).wait()

            transpose(b)
            writeback(u, b).start()
        return carry

    lax.fori_loop(0, U_PER_W // 2, body, 0)
    for i in (U_PER_W - 2, U_PER_W - 1):
        writeback(u0 + i, i % 2).wait()


@jax.jit
def _embedding_sc(idx5, weight_rows):
    mesh = plsc.VectorSubcoreMesh(core_axis_name="c", subcore_axis_name="s")
    f = pl.kernel(
        _emb_body,
        mesh=mesh,
        out_type=jax.ShapeDtypeStruct((HIST, 4, BT, 8, 128), jnp.float32),
        scratch_types=[
            pltpu.VMEM((4, ROWS), jnp.int32),
            pltpu.VMEM((ROWS, DIM), jnp.float32),
            pltpu.VMEM((ROWS, DIM), jnp.float32),
            pltpu.VMEM((4, 4, 8, 129), jnp.float32),
            pltpu.VMEM((4, 4, 8, 129), jnp.float32),
            pltpu.SemaphoreType.DMA((4,)),
            pltpu.SemaphoreType.DMA((2,)),
            pltpu.SemaphoreType.DMA((2,)),
        ],
        compiler_params=pltpu.CompilerParams(use_tc_tiling_on_sc=False,
                                             needs_layout_passes=False),
    )
    return f(idx5, weight_rows)


def kernel(input_, weight):
    idx5 = (input_.astype(jnp.int32)
            .reshape(BT, 128, HT, 8)
            .transpose(2, 0, 3, 1)
            .reshape(HT, BT, 1024))
    w128 = lax.optimization_barrier(jnp.reshape(weight, (NUM_EMB // 4, 128)))
    w_rows = jnp.reshape(w128, (NUM_EMB, DIM))
    out5 = _embedding_sc(idx5, w_rows)
    return out5.transpose(2, 4, 0, 1, 3).reshape(BATCH, HIST, DIM)


# speedup vs baseline: 3.8684x; 1.0241x over previous
import functools

import jax
import jax.numpy as jnp
from jax import lax
from jax.experimental import pallas as pl
from jax.experimental.pallas import tpu as pltpu
from jax.experimental.pallas import tpu_sc as plsc

NUM_EMB = 1000000
DIM = 32
BATCH = 16384
HIST = 200
NW = 32
HT = HIST // 8
BT = BATCH // 128
UNITS = HT * BT * 2
U_PER_W = UNITS // NW
ROWS = 512


def _emb_body(idx5_hbm, table_hbm, out5_hbm, idx_v, rows_v0, rows_v1, rows_v2,
              tile_v0, tile_v1, tile_v2, sem_i, sem_g, sem_o):
    rows_ring = (rows_v0, rows_v1, rows_v2)
    tile_ring = (tile_v0, tile_v1, tile_v2)
    c = lax.axis_index("c")
    s = lax.axis_index("s")
    wid = s * 2 + c
    u0 = wid * U_PER_W

    def decode(u):
        return u // 256, (u // 2) % 128, u % 2

    def idx_copy(u, ib):
        ht, bt, half = decode(u)
        return pltpu.make_async_copy(
            idx5_hbm.at[ht, bt, pl.ds(half * ROWS, ROWS)],
            idx_v.at[ib], sem_i.at[ib])

    def gather(ib, rb):
        return pltpu.make_async_copy(
            table_hbm.at[idx_v.at[ib]], rows_ring[rb], sem_g.at[rb])

    def writeback(u, rb):
        ht, bt, half = decode(u)
        return pltpu.make_async_copy(
            tile_ring[rb].at[:, :, :, pl.ds(0, 128)],
            out5_hbm.at[pl.ds(ht * 8 + half * 4, 4), :, bt],
            sem_o.at[rb])

    iota16 = lax.iota(jnp.int32, 16)
    zero16 = jnp.zeros((16,), jnp.int32)
    dtv = [(iota16 + q * 16) // 8 for q in range(2)]
    ddv = [(iota16 + q * 16) % 8 for q in range(2)]

    def transpose(rb):
        @plsc.parallel_loop(0, 128, unroll=8)
        def _(bb):
            bbv = zero16 + bb
            for hh in range(4):
                for q in range(2):
                    v = rows_ring[rb][hh * 128 + bb, pl.ds(q * 16, 16)]
                    plsc.store_scatter(tile_ring[rb].at[hh],
                                       [dtv[q], ddv[q], bbv], v)

    for k in range(4):
        idx_copy(u0 + k, k).start()
    idx_copy(u0, 0).wait()
    gather(0, 0).start()
    idx_copy(u0 + 1, 1).wait()
    gather(1, 1).start()

    def step(i, b):
        u = u0 + i
        gather(lax.rem(i, 4), b).wait()

        @pl.when(i + 4 < U_PER_W)
        def _():
            idx_copy(u + 4, lax.rem(i, 4)).start()

        @pl.when(i + 2 < U_PER_W)
        def _():
            idx_copy(u + 2, lax.rem(i + 2, 4)).wait()
            gather(lax.rem(i + 2, 4), (b + 2) % 3).start()

        @pl.when(i >= 3)
        def _():
            writeback(u - 3, b).wait()

        transpose(b)
        writeback(u, b).start()

    def body(p, carry):
        for b in (0, 1, 2):
            step(p * 3 + b, b)
        return carry

    lax.fori_loop(0, (U_PER_W - 2) // 3, body, 0)
    for i in (U_PER_W - 2, U_PER_W - 1):
        step(i, i % 3)
    for i in (U_PER_W - 3, U_PER_W - 2, U_PER_W - 1):
        writeback(u0 + i, i % 3).wait()


@jax.jit
def _embedding_sc(idx5, weight_rows):
    mesh = plsc.VectorSubcoreMesh(core_axis_name="c", subcore_axis_name="s")
    f = pl.kernel(
        _emb_body,
        mesh=mesh,
        out_type=jax.ShapeDtypeStruct((HIST, 4, BT, 8, 128), jnp.float32),
        scratch_types=[
            pltpu.VMEM((4, ROWS), jnp.int32),
            pltpu.VMEM((ROWS, DIM), jnp.float32),
            pltpu.VMEM((ROWS, DIM), jnp.float32),
            pltpu.VMEM((ROWS, DIM), jnp.float32),
            pltpu.VMEM((4, 4, 8, 129), jnp.float32),
            pltpu.VMEM((4, 4, 8, 129), jnp.float32),
            pltpu.VMEM((4, 4, 8, 129), jnp.float32),
            pltpu.SemaphoreType.DMA((4,)),
            pltpu.SemaphoreType.DMA((3,)),
            pltpu.SemaphoreType.DMA((3,)),
        ],
        compiler_params=pltpu.CompilerParams(use_tc_tiling_on_sc=False,
                                             needs_layout_passes=False),
    )
    return f(idx5, weight_rows)


def kernel(input_, weight):
    idx5 = (input_.astype(jnp.int32)
            .reshape(BT, 128, HT, 8)
            .transpose(2, 0, 3, 1)
            .reshape(HT, BT, 1024))
    w128 = lax.optimization_barrier(jnp.reshape(weight, (NUM_EMB // 4, 128)))
    w_rows = jnp.reshape(w128, (NUM_EMB, DIM))
    out5 = _embedding_sc(idx5, w_rows)
    return out5.transpose(2, 4, 0, 1, 3).reshape(BATCH, HIST, DIM)
